# baseline (speedup 1.0000x reference)
import functools

import jax
import jax.numpy as jnp
from jax import lax
from jax.experimental import pallas as pl
from jax.experimental.pallas import tpu as pltpu
from jax.experimental.pallas import tpu_sc as plsc

NC = 2
NS = 16
NW = NC * NS

C1 = 40
C2 = 40
_MESH = dict(core_axis_name="c", subcore_axis_name="s", num_cores=NC,
             num_subcores=NS)


def _phase1(edges, i0r, n_nodes):
    E, D = edges.shape
    per_tile = E // NW
    n_chunks = per_tile // C1
    EC = 40
    n_pool_chunks = n_nodes // EC
    nsub = D // 16

    @functools.partial(
        pl.kernel,
        out_type=jax.ShapeDtypeStruct((NC, n_nodes, D), jnp.float32),
        mesh=plsc.VectorSubcoreMesh(**_MESH),
        scratch_types=[
            pltpu.VMEM_SHARED((n_nodes, D), jnp.float32),
            pltpu.VMEM((C1, D), jnp.float32),
            pltpu.VMEM((C1, D), jnp.float32),
            pltpu.VMEM((C1, D), jnp.float32),
            pltpu.VMEM((n_chunks, C1), jnp.int32),
            pltpu.SemaphoreType.DMA,
            pltpu.SemaphoreType.DMA,
            pltpu.SemaphoreType.DMA,
            pltpu.SemaphoreType.DMA,
            pltpu.SemaphoreType.DMA,
            pltpu.SemaphoreType.DMA,
        ],
    )
    def k1(edges_hbm, i0_hbm, out_hbm, pool_sh,
           r0_, r1_, r2_, idx_all,
           ld0, ld1, ld2, sa0, sa1, sa2):
        exp_v = r0_
        c = lax.axis_index("c")
        s = lax.axis_index("s")
        wid = c * NS + s
        base = wid * per_tile

        pltpu.sync_copy(i0_hbm.at[wid], idx_all)

        zero = jnp.zeros((16,), jnp.float32)

        def zbody(i, carry):
            r = i // nsub
            j = (i % nsub) * 16
            exp_v[r, pl.ds(j, 16)] = zero
            return carry

        lax.fori_loop(0, EC * nsub, zbody, 0)

        def zchunk(kk, carry):
            @pl.when(kk % NS == s)
            def _():
                pltpu.sync_copy(exp_v, pool_sh.at[pl.ds(kk * EC, EC)])
            return carry

        lax.fori_loop(0, n_pool_chunks, zchunk, 0)
        plsc.subcore_barrier()

        rows = (r0_, r1_, r2_)
        ld = (ld0, ld1, ld2)
        sa = (sa0, sa1, sa2)

        def load(k, b):
            pltpu.async_copy(edges_hbm.at[pl.ds(base + k * C1, C1)],
                             rows[b], ld[b])

        def wait_load(k, b):
            pltpu.make_async_copy(edges_hbm.at[pl.ds(base + k * C1, C1)],
                                  rows[b], ld[b]).wait()

        def scat(k, b):
            pltpu.async_copy(rows[b], pool_sh.at[idx_all.at[k]], sa[b],
                             add=True)

        def wait_scat(k, b):
            pltpu.make_async_copy(rows[b], pool_sh.at[idx_all.at[k]],
                                  sa[b]).wait()

        load(0, 0)

        def body3(q, carry):
            k0 = q * 3
            for b in (0, 1, 2):
                k = k0 + b
                bn = (b + 1) % 3

                @pl.when(k >= 2)
                def _():
                    wait_scat(k - 2, bn)

                @pl.when(k + 1 < n_chunks)
                def _():
                    load(k + 1, bn)

                wait_load(k, b)
                scat(k, b)
            return carry

        lax.fori_loop(0, n_chunks // 3, body3, 0)
        for k in range(n_chunks - n_chunks % 3, n_chunks):
            b = k % 3
            bn = (b + 1) % 3
            wait_scat(k - 2, bn)
            wait_load(k, b)
            scat(k, b)
        wait_scat(n_chunks - 2, (n_chunks - 2) % 3)
        wait_scat(n_chunks - 1, (n_chunks - 1) % 3)
        plsc.subcore_barrier()

        def echunk(kk, carry):
            @pl.when(kk % NS == s)
            def _():
                r0 = kk * EC
                pltpu.sync_copy(pool_sh.at[pl.ds(r0, EC)], exp_v)
                pltpu.sync_copy(exp_v, out_hbm.at[c, pl.ds(r0, EC)])
            return carry

        lax.fori_loop(0, n_pool_chunks, echunk, 0)

    return k1(edges, i0r)


def _combine(partials):
    _, N, D = partials.shape
    BLK = 1000

    def body(p0_ref, p1_ref, o_ref):
        o_ref[...] = p0_ref[...] + p1_ref[...]

    return pl.pallas_call(
        body,
        grid=(N // BLK,),
        in_specs=[pl.BlockSpec((BLK, D), lambda i: (i, 0)),
                  pl.BlockSpec((BLK, D), lambda i: (i, 0))],
        out_specs=pl.BlockSpec((BLK, D), lambda i: (i, 0)),
        out_shape=jax.ShapeDtypeStruct((N, D), jnp.float32),
    )(partials[0], partials[1])


def _phase2(pool, edges, i1r, epr):
    E, D = edges.shape
    per_tile = E // NW
    n_chunks = per_tile // C2
    nsub = D // 16

    @functools.partial(
        pl.kernel,
        out_type=jax.ShapeDtypeStruct((E, D), jnp.float32),
        mesh=plsc.VectorSubcoreMesh(**_MESH),
        scratch_types=[
            pltpu.VMEM((C2, D), jnp.float32),
            pltpu.VMEM((C2, D), jnp.float32),
            pltpu.VMEM((C2, D), jnp.float32),
            pltpu.VMEM((C2, D), jnp.float32),
            pltpu.VMEM((C2, D), jnp.float32),
            pltpu.VMEM((C2, D), jnp.float32),
            pltpu.VMEM((C2, D), jnp.float32),
            pltpu.VMEM((C2, D), jnp.float32),
            pltpu.VMEM((n_chunks, C2), jnp.int32),
            pltpu.VMEM((n_chunks, C2), jnp.int32),
            pltpu.SemaphoreType.DMA,
            pltpu.SemaphoreType.DMA,
            pltpu.SemaphoreType.DMA,
            pltpu.SemaphoreType.DMA,
            pltpu.SemaphoreType.DMA,
            pltpu.SemaphoreType.DMA,
        ],
    )
    def k2(pool_hbm, edges_hbm, i1_hbm, ep_hbm, out_hbm,
           a0, a1, a2, b0, b1, b2, o0, o1, idx1_all, idxp_all,
           g0, g1, g2, st0, st1, st2):
        c = lax.axis_index("c")
        s = lax.axis_index("s")
        wid = c * NS + s
        base = wid * per_tile

        pltpu.sync_copy(i1_hbm.at[wid], idx1_all)
        pltpu.sync_copy(ep_hbm.at[wid], idxp_all)

        A = (a0, a1, a2)
        B = (b0, b1, b2)
        O = (o0, o1)
        g = (g0, g1, g2)
        st = (st0, st1)

        def gathers(k, b):
            pltpu.async_copy(pool_hbm.at[idx1_all.at[k]], A[b], g[b])
            pltpu.async_copy(edges_hbm.at[idxp_all.at[k]], B[b], g[b])

        def wait_gathers(k, b):
            pltpu.make_async_copy(pool_hbm.at[idx1_all.at[k]], A[b],
                                  g[b]).wait()
            pltpu.make_async_copy(edges_hbm.at[idxp_all.at[k]], B[b],
                                  g[b]).wait()

        def store(k, bo):
            pltpu.async_copy(O[bo], out_hbm.at[pl.ds(base + k * C2, C2)],
                             st[bo])

        def wait_store(k, bo):
            pltpu.make_async_copy(O[bo],
                                  out_hbm.at[pl.ds(base + k * C2, C2)],
                                  st[bo]).wait()

        def subtract(b, bo):
            ab, bb, ob = A[b], B[b], O[bo]

            def sbody(r, carry):
                for jj in range(nsub):
                    j = jj * 16
                    ob[r, pl.ds(j, 16)] = (ab[r, pl.ds(j, 16)]
                                           - bb[r, pl.ds(j, 16)])
                return carry

            lax.fori_loop(0, C2, sbody, 0)

        gathers(0, 0)
        gathers(1, 1)
        gathers(2, 2)

        def body6(q, carry):
            k0 = q * 6
            for b6 in range(6):
                k = k0 + b6
                b = b6 % 3
                bo = b6 % 2
                wait_gathers(k, b)

                @pl.when(k >= 2)
                def _():
                    wait_store(k - 2, bo)

                subtract(b, bo)
                store(k, bo)

                @pl.when(k + 3 < n_chunks)
                def _():
                    gathers(k + 3, b)
            return carry

        lax.fori_loop(0, n_chunks // 6, body6, 0)
        for k in range(n_chunks - n_chunks % 6, n_chunks):
            b = k % 3
            bo = k % 2
            wait_gathers(k, b)
            wait_store(k - 2, k % 2)
            subtract(b, bo)
            store(k, bo)
            if k + 3 < n_chunks:
                gathers(k + 3, b)
        wait_store(n_chunks - 2, n_chunks % 2)
        wait_store(n_chunks - 1, (n_chunks - 1) % 2)

    return k2(pool, edges, i1r, epr)


def kernel(nodes, edges, edge_index, edge_pair):
    n_nodes = nodes.shape[0]
    E = edges.shape[0]
    per_tile = E // NW
    i0r = edge_index[0].reshape(NW, per_tile // C1, C1)
    i1r = edge_index[1].reshape(NW, per_tile // C2, C2)
    epr = edge_pair[0].reshape(NW, per_tile // C2, C2)
    partials = _phase1(edges, i0r, n_nodes)
    pool = _combine(partials)
    return _phase2(pool, edges, i1r, epr)


# speedup vs baseline: 1.0459x; 1.0459x over previous
import functools

import jax
import jax.numpy as jnp
from jax import lax
from jax.experimental import pallas as pl
from jax.experimental.pallas import tpu as pltpu
from jax.experimental.pallas import tpu_sc as plsc

NC = 2
NS = 16
NW = NC * NS

C = 80
_MESH = dict(core_axis_name="c", subcore_axis_name="s", num_cores=NC,
             num_subcores=NS)


def _phase1(edges, i0r, n_nodes):
    E, D = edges.shape
    per_tile = E // NW
    n_chunks = per_tile // C
    EC = 80
    n_pool_chunks = n_nodes // EC
    nsub = D // 16

    @functools.partial(
        pl.kernel,
        out_type=jax.ShapeDtypeStruct((NC, n_nodes, D), jnp.float32),
        mesh=plsc.VectorSubcoreMesh(**_MESH),
        scratch_types=[
            pltpu.VMEM_SHARED((n_nodes, D), jnp.float32),
            pltpu.VMEM((C, D), jnp.float32),
            pltpu.VMEM((C, D), jnp.float32),
            pltpu.VMEM((n_chunks, C), jnp.int32),
            pltpu.VMEM((EC, D), jnp.float32),
            pltpu.SemaphoreType.DMA,
            pltpu.SemaphoreType.DMA,
            pltpu.SemaphoreType.DMA,
            pltpu.SemaphoreType.DMA,
        ],
    )
    def k1(edges_hbm, i0_hbm, out_hbm, pool_sh, rows0, rows1, idx_all, exp_v,
           ld0, ld1, sa0, sa1):
        c = lax.axis_index("c")
        s = lax.axis_index("s")
        wid = c * NS + s
        base = wid * per_tile

        pltpu.sync_copy(i0_hbm.at[wid], idx_all)

        zero = jnp.zeros((16,), jnp.float32)

        def zbody(i, carry):
            r = i // nsub
            j = (i % nsub) * 16
            exp_v[r, pl.ds(j, 16)] = zero
            return carry

        lax.fori_loop(0, EC * nsub, zbody, 0)

        def zchunk(kk, carry):
            @pl.when(kk % NS == s)
            def _():
                pltpu.sync_copy(exp_v, pool_sh.at[pl.ds(kk * EC, EC)])
            return carry

        lax.fori_loop(0, n_pool_chunks, zchunk, 0)
        plsc.subcore_barrier()

        rows = (rows0, rows1)
        ld = (ld0, ld1)
        sa = (sa0, sa1)

        def load(k, b):
            return pltpu.async_copy(
                edges_hbm.at[pl.ds(base + k * C, C)], rows[b], ld[b])

        load(0, 0)
        load(1, 1)

        def body2(g, carry):
            k0 = g * 2
            for b in (0, 1):
                k = k0 + b
                pltpu.make_async_copy(
                    edges_hbm.at[pl.ds(base + k * C, C)], rows[b],
                    ld[b]).wait()
                cp = pltpu.async_copy(rows[b], pool_sh.at[idx_all.at[k]],
                                      sa[b], add=True)
                cp.wait()

                @pl.when(k + 2 < n_chunks)
                def _():
                    load(k + 2, b)
            return carry

        lax.fori_loop(0, n_chunks // 2, body2, 0)
        if n_chunks % 2:
            k = n_chunks - 1
            b = k % 2
            pltpu.make_async_copy(
                edges_hbm.at[pl.ds(base + k * C, C)], rows[b], ld[b]).wait()
            pltpu.async_copy(rows[b], pool_sh.at[idx_all.at[k]], sa[b],
                             add=True).wait()
        plsc.subcore_barrier()

        def echunk(kk, carry):
            @pl.when(kk % NS == s)
            def _():
                r0 = kk * EC
                pltpu.sync_copy(pool_sh.at[pl.ds(r0, EC)], exp_v)
                pltpu.sync_copy(exp_v, out_hbm.at[c, pl.ds(r0, EC)])
            return carry

        lax.fori_loop(0, n_pool_chunks, echunk, 0)

    return k1(edges, i0r)


def _combine(partials):
    _, N, D = partials.shape
    BLK = 1000

    def body(p0_ref, p1_ref, o_ref):
        o_ref[...] = p0_ref[...] + p1_ref[...]

    return pl.pallas_call(
        body,
        grid=(N // BLK,),
        in_specs=[pl.BlockSpec((BLK, D), lambda i: (i, 0)),
                  pl.BlockSpec((BLK, D), lambda i: (i, 0))],
        out_specs=pl.BlockSpec((BLK, D), lambda i: (i, 0)),
        out_shape=jax.ShapeDtypeStruct((N, D), jnp.float32),
    )(partials[0], partials[1])


def _phase2(pool, edges, i1r, epr):
    E, D = edges.shape
    per_tile = E // NW
    n_chunks = per_tile // C
    nsub = D // 16

    @functools.partial(
        pl.kernel,
        out_type=jax.ShapeDtypeStruct((E, D), jnp.float32),
        mesh=plsc.VectorSubcoreMesh(**_MESH),
        scratch_types=[
            pltpu.VMEM((C, D), jnp.float32),
            pltpu.VMEM((C, D), jnp.float32),
            pltpu.VMEM((C, D), jnp.float32),
            pltpu.VMEM((C, D), jnp.float32),
            pltpu.VMEM((C, D), jnp.float32),
            pltpu.VMEM((C, D), jnp.float32),
            pltpu.VMEM((n_chunks, C), jnp.int32),
            pltpu.VMEM((n_chunks, C), jnp.int32),
            pltpu.SemaphoreType.DMA,
            pltpu.SemaphoreType.DMA,
            pltpu.SemaphoreType.DMA,
            pltpu.SemaphoreType.DMA,
        ],
    )
    def k2(pool_hbm, edges_hbm, i1_hbm, ep_hbm, out_hbm,
           a0, a1, b0, b1, o0, o1, idx1_all, idxp_all, g0, g1, st0, st1):
        c = lax.axis_index("c")
        s = lax.axis_index("s")
        wid = c * NS + s
        base = wid * per_tile

        pltpu.sync_copy(i1_hbm.at[wid], idx1_all)
        pltpu.sync_copy(ep_hbm.at[wid], idxp_all)

        A = (a0, a1)
        B = (b0, b1)
        O = (o0, o1)
        g = (g0, g1)
        st = (st0, st1)

        def gathers(k, b):
            pltpu.async_copy(pool_hbm.at[idx1_all.at[k]], A[b], g[b])
            pltpu.async_copy(edges_hbm.at[idxp_all.at[k]], B[b], g[b])

        def wait_gathers(k, b):
            pltpu.make_async_copy(pool_hbm.at[idx1_all.at[k]], A[b],
                                  g[b]).wait()
            pltpu.make_async_copy(edges_hbm.at[idxp_all.at[k]], B[b],
                                  g[b]).wait()

        def store(k, b):
            pltpu.async_copy(O[b], out_hbm.at[pl.ds(base + k * C, C)],
                             st[b])

        def wait_store(k, b):
            pltpu.make_async_copy(O[b], out_hbm.at[pl.ds(base + k * C, C)],
                                  st[b]).wait()

        def subtract(b):
            ab, bb, ob = A[b], B[b], O[b]

            def sbody(r, carry):
                for jj in range(nsub):
                    j = jj * 16
                    ob[r, pl.ds(j, 16)] = (ab[r, pl.ds(j, 16)]
                                           - bb[r, pl.ds(j, 16)])
                return carry

            lax.fori_loop(0, C, sbody, 0)

        gathers(0, 0)
        gathers(1, 1)

        def body2(gidx, carry):
            k0 = gidx * 2
            for b in (0, 1):
                k = k0 + b
                wait_gathers(k, b)

                @pl.when(k > 1)
                def _():
                    wait_store(k - 2, b)

                subtract(b)
                store(k, b)

                @pl.when(k + 2 < n_chunks)
                def _():
                    gathers(k + 2, b)
            return carry

        lax.fori_loop(0, n_chunks // 2, body2, 0)
        if n_chunks % 2:
            k = n_chunks - 1
            b = k % 2
            wait_gathers(k, b)
            wait_store(k - 2, b)
            subtract(b)
            store(k, b)
            wait_store(k - 1, 1 - b)
            wait_store(k, b)
        else:
            wait_store(n_chunks - 2, 0)
            wait_store(n_chunks - 1, 1)

    return k2(pool, edges, i1r, epr)


def kernel(nodes, edges, edge_index, edge_pair):
    n_nodes = nodes.shape[0]
    E = edges.shape[0]
    per_tile = E // NW
    n_chunks = per_tile // C
    i0r = edge_index[0].reshape(NW, n_chunks, C)
    i1r = edge_index[1].reshape(NW, n_chunks, C)
    epr = edge_pair[0].reshape(NW, n_chunks, C)
    partials = _phase1(edges, i0r, n_nodes)
    pool = _combine(partials)
    return _phase2(pool, edges, i1r, epr)
